# baseline (speedup 1.0000x reference)
import functools

import jax
import jax.numpy as jnp
from jax.experimental import pallas as pl
from jax.experimental.pallas import tpu as pltpu

N = 5000
NP = 5120
K_PAIRS = 4096
TM = 256


def _pairs_body(s_ref, ns_ref, o_ref, no_ref, out_ref):
    s = s_ref[...]
    ns = ns_ref[...]
    o = o_ref[...]
    no = no_ref[...]
    dn = (((1,), (1,)), ((), ()))
    out_ref[...] = (jax.lax.dot_general(s, o, dn) +
                    jax.lax.dot_general(ns, no, dn))


def _pair_logits(s, ns, o, no):
    grid = (NP // TM,)
    return pl.pallas_call(
        _pairs_body,
        grid=grid,
        in_specs=[
            pl.BlockSpec((TM, 64), lambda i: (i, 0)),
            pl.BlockSpec((TM, 64), lambda i: (i, 0)),
            pl.BlockSpec((NP, 64), lambda i: (0, 0)),
            pl.BlockSpec((NP, 64), lambda i: (0, 0)),
        ],
        out_specs=pl.BlockSpec((TM, NP), lambda i: (i, 0)),
        out_shape=jax.ShapeDtypeStruct((NP, NP), jnp.float32),
    )(s, ns, o, no)


def kernel(rois, roi_feat, nlp_feat, im_info, gt_boxes, gt_relation, num_boxes,
           W_sub, W_obj, U_sub, U_obj):
    rf = roi_feat[0]
    nf = nlp_feat[0]
    s = rf @ W_sub
    o = rf @ W_obj
    ns = nf @ U_sub
    no = nf @ U_obj
    pad0 = ((0, NP - N), (0, 0))
    logits_p = _pair_logits(jnp.pad(s, pad0), jnp.pad(ns, pad0),
                            jnp.pad(o, pad0), jnp.pad(no, pad0))
    r = jnp.arange(NP)
    bad = (r[:, None] == r[None, :]) | (r[:, None] >= N) | (r[None, :] >= N)
    scores = jnp.where(bad, 0.0, jax.nn.sigmoid(logits_p))
    topv, topi = jax.lax.top_k(scores.reshape(-1), K_PAIRS)
    i = topi // NP
    j = topi % NP
    rois0 = rois[0]
    bidx = rois0[i, 0:1]
    boxes_i = rois0[i, 1:5]
    boxes_j = rois0[j, 1:5]
    pairs = jnp.concatenate([bidx, boxes_i, boxes_j], axis=1)[None]
    props = jnp.stack([i, j], axis=1)[None]
    relpn_loss_cls = jnp.array(0.0, dtype=jnp.float32)
    relpn_eval = jnp.zeros((3,), dtype=jnp.float32)
    return (pairs, props, topv[None], relpn_loss_cls, relpn_eval)


# speedup vs baseline: 5.3404x; 5.3404x over previous
import functools

import jax
import jax.numpy as jnp
from jax import lax
from jax.experimental import pallas as pl
from jax.experimental.pallas import tpu as pltpu
from jax.experimental.pallas import tpu_sc as plsc
from jax.scipy.special import ndtri

N = 5000
NP = 5120
K_PAIRS = 4096
TM = 256

NW = 32
WORDS_W = NP * NP // 4 // NW
CHUNK_W = 25600
NCHUNK = WORDS_W // CHUNK_W
NZ_CAP = 2560
NZ_PAD = NZ_CAP + 16
CAND_CAP = 2176
CAND_PAD = CAND_CAP + 16
TARGET = 16384.0


def _score_body(thr_ref, s_ref, ns_ref, o_ref, no_ref, out_ref, mask_ref):
    i = pl.program_id(0)
    dn = (((1,), (1,)), ((), ()))
    logits = (lax.dot_general(s_ref[...], o_ref[...], dn) +
              lax.dot_general(ns_ref[...], no_ref[...], dn))
    out_ref[...] = logits
    thr = thr_ref[0]
    rows = lax.broadcasted_iota(jnp.int32, (TM, NP), 0) + i * TM
    cols = lax.broadcasted_iota(jnp.int32, (TM, NP), 1)
    ok = (logits >= thr) & (cols != rows) & (cols < N) & (rows < N)
    mask_ref[...] = ok.astype(jnp.int8)


def _score_call(thr, s, ns, o, no):
    return pl.pallas_call(
        _score_body,
        grid=(NP // TM,),
        in_specs=[
            pl.BlockSpec(memory_space=pltpu.SMEM),
            pl.BlockSpec((TM, 64), lambda i: (i, 0)),
            pl.BlockSpec((TM, 64), lambda i: (i, 0)),
            pl.BlockSpec((NP, 64), lambda i: (0, 0)),
            pl.BlockSpec((NP, 64), lambda i: (0, 0)),
        ],
        out_specs=(
            pl.BlockSpec((TM, NP), lambda i: (i, 0)),
            pl.BlockSpec((TM, NP), lambda i: (i, 0)),
        ),
        out_shape=(
            jax.ShapeDtypeStruct((NP, NP), jnp.float32),
            jax.ShapeDtypeStruct((NP, NP), jnp.int8),
        ),
    )(thr, s, ns, o, no)


def _extract_call(words, lg_flat):
    mesh = plsc.VectorSubcoreMesh(core_axis_name="c", subcore_axis_name="s")

    @functools.partial(
        pl.kernel,
        out_type=(
            jax.ShapeDtypeStruct((NW, CAND_PAD), jnp.int32),
            jax.ShapeDtypeStruct((NW, CAND_PAD), jnp.float32),
            jax.ShapeDtypeStruct((NW, 16), jnp.int32),
        ),
        mesh=mesh,
        compiler_params=pltpu.CompilerParams(needs_layout_passes=False),
        scratch_types=[
            pltpu.VMEM((CHUNK_W,), jnp.int32),
            pltpu.VMEM((NZ_PAD,), jnp.int32),
            pltpu.VMEM((NZ_PAD,), jnp.int32),
            pltpu.VMEM((CAND_PAD,), jnp.int32),
            pltpu.VMEM((CAND_PAD,), jnp.float32),
            pltpu.VMEM((16,), jnp.int32),
            pltpu.SemaphoreType.DMA,
        ],
    )
    def k(words_hbm, lg_hbm, cand_hbm, vals_hbm, cnt_hbm,
          wbuf, nzw, nzv, cand, vals, cnt_v, sem):
        w = lax.axis_index("s") * 2 + lax.axis_index("c")
        base_word = w * WORDS_W
        iota = lax.iota(jnp.int32, 16)

        @pl.loop(0, CAND_PAD, step=16)
        def _(p):
            cand[pl.ds(p, 16)] = jnp.zeros((16,), jnp.int32)

        def chunk(c, off):
            pltpu.sync_copy(
                words_hbm.at[pl.ds(base_word + c * CHUNK_W, CHUNK_W)], wbuf)

            def grp(g, off):
                w16 = wbuf[pl.ds(g * 16, 16)]
                m = w16 != 0
                mi = m.astype(jnp.int32)
                nz = jnp.sum(mi)
                pos = jnp.minimum(off + plsc.cumsum(mi) - 1, NZ_PAD - 1)
                gid = base_word + c * CHUNK_W + g * 16 + iota
                plsc.store_scatter(nzw, [pos], gid, mask=m)
                plsc.store_scatter(nzv, [pos], w16, mask=m)
                return off + nz

            return lax.fori_loop(0, CHUNK_W // 16, grp, off)

        nz_total = lax.fori_loop(0, NCHUNK, chunk, jnp.int32(0))
        nz_lim = jnp.minimum(nz_total, NZ_CAP)

        def grpb(g, coff):
            ids = nzw[pl.ds(g * 16, 16)]
            wv = nzv[pl.ds(g * 16, 16)]
            lane_ok = (g * 16 + iota) < nz_lim
            bmis = []
            bms = []
            for bi in range(4):
                byte = (wv >> (8 * bi)) & 255
                bm = (byte != 0) & lane_ok
                bms.append(bm)
                bmis.append(bm.astype(jnp.int32))
            pw = bmis[0] + bmis[1] + bmis[2] + bmis[3]
            wpre = plsc.cumsum(pw) - pw
            run = jnp.zeros((16,), jnp.int32)
            for bi in range(4):
                pos = jnp.minimum(coff + wpre + run, CAND_PAD - 1)
                plsc.store_scatter(cand, [pos], ids * 4 + bi, mask=bms[bi])
                run = run + bmis[bi]
            return coff + jnp.sum(pw)

        cand_total = lax.fori_loop(0, NZ_PAD // 16, grpb, jnp.int32(0))

        pltpu.async_copy(lg_hbm.at[cand], vals, sem).wait()

        pltpu.sync_copy(cand, cand_hbm.at[w])
        pltpu.sync_copy(vals, vals_hbm.at[w])
        big = jnp.int32(1 << 30)
        cnt_v[...] = jnp.where(
            iota == 1,
            jnp.full((16,), jnp.minimum(nz_total, big), jnp.int32),
            jnp.full((16,), jnp.minimum(cand_total, big), jnp.int32))
        pltpu.sync_copy(cnt_v, cnt_hbm.at[w])

    return k(words, lg_flat)


def kernel(rois, roi_feat, nlp_feat, im_info, gt_boxes, gt_relation, num_boxes,
           W_sub, W_obj, U_sub, U_obj):
    rf = roi_feat[0]
    nf = nlp_feat[0]
    s = rf @ W_sub
    o = rf @ W_obj
    ns = nf @ U_sub
    no = nf @ U_obj

    A = jnp.concatenate([s, ns], axis=1)
    Bm = jnp.concatenate([o, no], axis=1)
    mu = (A.mean(0) @ Bm.mean(0))
    ex2 = jnp.sum((A.T @ A) * (Bm.T @ Bm)) / (N * N)
    sig = jnp.sqrt(jnp.maximum(ex2 - mu * mu, 1e-12))
    z = ndtri(jnp.float32(1.0 - TARGET / (N * N)))
    thr = mu + sig * z
    thr_cert = thr + 2e-3 * jnp.maximum(1.0, jnp.abs(thr))

    pad0 = ((0, NP - N), (0, 0))
    logits, maskb = _score_call(
        thr[None], jnp.pad(s, pad0), jnp.pad(ns, pad0),
        jnp.pad(o, pad0), jnp.pad(no, pad0))

    words = lax.bitcast_convert_type(maskb.reshape(-1, 4), jnp.int32)
    lg_flat = logits.reshape(-1)
    cand, vals, cnts = _extract_call(words, lg_flat)

    counts = cnts[:, 0]
    nzs = cnts[:, 1]
    slot = jnp.arange(CAND_PAD)[None, :]
    valid = (slot < jnp.minimum(counts, CAND_CAP)[:, None]).reshape(-1)
    vflat = vals.reshape(-1)
    cflat = cand.reshape(-1)
    sv = jnp.where(valid, jax.nn.sigmoid(vflat), -1.0)
    cert = jnp.sum((valid & (vflat >= thr_cert)).astype(jnp.int32))
    ok = ((cert >= K_PAIRS) & jnp.all(counts <= CAND_CAP)
          & jnp.all(nzs <= NZ_CAP))

    rois0 = rois[0]

    def finish(idx, topv):
        i = idx // NP
        j = idx % NP
        bidx = rois0[i, 0:1]
        boxes_i = rois0[i, 1:5]
        boxes_j = rois0[j, 1:5]
        pairs = jnp.concatenate([bidx, boxes_i, boxes_j], axis=1)
        props = jnp.stack([i, j], axis=1)
        return pairs, props, topv

    def fast():
        topv, pos = lax.top_k(sv, K_PAIRS)
        return finish(cflat[pos], topv)

    def slow():
        r = jnp.arange(NP)
        bad = ((r[:, None] == r[None, :]) | (r[:, None] >= N)
               | (r[None, :] >= N))
        scores = jnp.where(bad, 0.0, jax.nn.sigmoid(logits))
        topv, topi = lax.top_k(scores.reshape(-1), K_PAIRS)
        return finish(topi, topv)

    pairs, props, topv = lax.cond(ok, fast, slow)
    relpn_loss_cls = jnp.array(0.0, dtype=jnp.float32)
    relpn_eval = jnp.zeros((3,), dtype=jnp.float32)
    return (pairs[None], props[None], topv[None], relpn_loss_cls, relpn_eval)


# speedup vs baseline: 58.6794x; 10.9877x over previous
import functools

import jax
import jax.numpy as jnp
from jax import lax
from jax.experimental import pallas as pl
from jax.experimental.pallas import tpu as pltpu
from jax.experimental.pallas import tpu_sc as plsc
from jax.scipy.special import ndtri

N = 5000
NP = 5120
K_PAIRS = 4096
TM = 256
WPR = NP // 32
NWORDS = NP * NP // 32

NW = 32
WORDS_W = NWORDS // NW
NZ_CAP = 2176
NZ_PAD = NZ_CAP + 16
CAND_CAP = 2176
CAND_PAD = CAND_CAP + 16
TARGET = 16384.0


def _pack_weights():
    c = jnp.arange(NP)
    wc = c // 32
    b = c % 32
    onehot = (wc[:, None] == jnp.arange(WPR)[None, :]).astype(jnp.float32)
    p_lo = onehot * jnp.where(b < 16, 2.0 ** (b % 16), 0.0)[:, None]
    p_hi = onehot * jnp.where(b >= 16, 2.0 ** (b % 16), 0.0)[:, None]
    return p_lo, p_hi


def _score_body(thr_ref, s_ref, ns_ref, o_ref, no_ref, plo_ref, phi_ref,
                out_ref, words_ref):
    i = pl.program_id(0)
    dn = (((1,), (1,)), ((), ()))
    logits = (lax.dot_general(s_ref[...], o_ref[...], dn) +
              lax.dot_general(ns_ref[...], no_ref[...], dn))
    out_ref[...] = logits
    thr = thr_ref[0]
    rows = lax.broadcasted_iota(jnp.int32, (TM, NP), 0) + i * TM
    cols = lax.broadcasted_iota(jnp.int32, (TM, NP), 1)
    ok = (logits >= thr) & (cols != rows) & (cols < N) & (rows < N)
    ind = ok.astype(jnp.float32)
    dnn = (((1,), (0,)), ((), ()))
    lo = lax.dot_general(ind, plo_ref[...], dnn)
    hi = lax.dot_general(ind, phi_ref[...], dnn)
    words_ref[...] = lo.astype(jnp.int32) | (hi.astype(jnp.int32) << 16)


def _score_call(thr, s, ns, o, no, p_lo, p_hi):
    return pl.pallas_call(
        _score_body,
        grid=(NP // TM,),
        in_specs=[
            pl.BlockSpec(memory_space=pltpu.SMEM),
            pl.BlockSpec((TM, 64), lambda i: (i, 0)),
            pl.BlockSpec((TM, 64), lambda i: (i, 0)),
            pl.BlockSpec((NP, 64), lambda i: (0, 0)),
            pl.BlockSpec((NP, 64), lambda i: (0, 0)),
            pl.BlockSpec((NP, WPR), lambda i: (0, 0)),
            pl.BlockSpec((NP, WPR), lambda i: (0, 0)),
        ],
        out_specs=(
            pl.BlockSpec((TM, NP), lambda i: (i, 0)),
            pl.BlockSpec((TM, WPR), lambda i: (i, 0)),
        ),
        out_shape=(
            jax.ShapeDtypeStruct((NP, NP), jnp.float32),
            jax.ShapeDtypeStruct((NP, WPR), jnp.int32),
        ),
    )(thr, s, ns, o, no, p_lo, p_hi)


def _extract_call(words, lg_flat):
    mesh = plsc.VectorSubcoreMesh(core_axis_name="c", subcore_axis_name="s")

    @functools.partial(
        pl.kernel,
        out_type=(
            jax.ShapeDtypeStruct((NW, CAND_PAD), jnp.int32),
            jax.ShapeDtypeStruct((NW, CAND_PAD), jnp.float32),
            jax.ShapeDtypeStruct((NW, 16), jnp.int32),
        ),
        mesh=mesh,
        compiler_params=pltpu.CompilerParams(needs_layout_passes=False),
        scratch_types=[
            pltpu.VMEM((WORDS_W,), jnp.int32),
            pltpu.VMEM((NZ_PAD,), jnp.int32),
            pltpu.VMEM((NZ_PAD,), jnp.int32),
            pltpu.VMEM((CAND_PAD,), jnp.int32),
            pltpu.VMEM((CAND_PAD,), jnp.float32),
            pltpu.VMEM((16,), jnp.int32),
            pltpu.SemaphoreType.DMA,
        ],
    )
    def k(words_hbm, lg_hbm, cand_hbm, vals_hbm, cnt_hbm,
          wbuf, nzw, nzv, cand, vals, cnt_v, sem):
        w = lax.axis_index("s") * 2 + lax.axis_index("c")
        base_word = w * WORDS_W
        iota = lax.iota(jnp.int32, 16)

        @pl.loop(0, CAND_PAD, step=16)
        def _(p):
            cand[pl.ds(p, 16)] = jnp.zeros((16,), jnp.int32)

        pltpu.sync_copy(words_hbm.at[pl.ds(base_word, WORDS_W)], wbuf)

        def grp(g, off):
            w16 = wbuf[pl.ds(g * 16, 16)]
            m = w16 != 0
            mi = m.astype(jnp.int32)
            nz = jnp.sum(mi)
            pos = jnp.minimum(off + plsc.cumsum(mi) - 1, NZ_PAD - 1)
            gid = base_word + g * 16 + iota
            plsc.store_scatter(nzw, [pos], gid, mask=m)
            plsc.store_scatter(nzv, [pos], w16, mask=m)
            return off + nz

        nz_total = lax.fori_loop(0, WORDS_W // 16, grp, jnp.int32(0))
        nz_lim = jnp.minimum(nz_total, NZ_CAP)

        def grpb(g, coff):
            ids = nzw[pl.ds(g * 16, 16)]
            wv = nzv[pl.ds(g * 16, 16)]
            lane_ok = (g * 16 + iota) < nz_lim
            bmis = []
            bms = []
            pw = jnp.zeros((16,), jnp.int32)
            for bi in range(32):
                bit = (wv >> bi) & 1
                bm = (bit != 0) & lane_ok
                bms.append(bm)
                bmi = bm.astype(jnp.int32)
                bmis.append(bmi)
                pw = pw + bmi
            wpre = plsc.cumsum(pw) - pw
            run = jnp.zeros((16,), jnp.int32)
            for bi in range(32):
                pos = jnp.minimum(coff + wpre + run, CAND_PAD - 1)
                plsc.store_scatter(cand, [pos], ids * 32 + bi, mask=bms[bi])
                run = run + bmis[bi]
            return coff + jnp.sum(pw)

        ngrp = (nz_lim + 15) // 16
        cand_total = lax.fori_loop(0, ngrp, grpb, jnp.int32(0))

        pltpu.async_copy(lg_hbm.at[cand], vals, sem).wait()

        pltpu.sync_copy(cand, cand_hbm.at[w])
        pltpu.sync_copy(vals, vals_hbm.at[w])
        big = jnp.int32(1 << 30)
        cnt_v[...] = jnp.where(
            iota == 1,
            jnp.full((16,), jnp.minimum(nz_total, big), jnp.int32),
            jnp.full((16,), jnp.minimum(cand_total, big), jnp.int32))
        pltpu.sync_copy(cnt_v, cnt_hbm.at[w])

    return k(words, lg_flat)


def kernel(rois, roi_feat, nlp_feat, im_info, gt_boxes, gt_relation, num_boxes,
           W_sub, W_obj, U_sub, U_obj):
    rf = roi_feat[0]
    nf = nlp_feat[0]
    s = rf @ W_sub
    o = rf @ W_obj
    ns = nf @ U_sub
    no = nf @ U_obj

    A = jnp.concatenate([s, ns], axis=1)
    Bm = jnp.concatenate([o, no], axis=1)
    mu = (A.mean(0) @ Bm.mean(0))
    ex2 = jnp.sum((A.T @ A) * (Bm.T @ Bm)) / (N * N)
    sig = jnp.sqrt(jnp.maximum(ex2 - mu * mu, 1e-12))
    z = ndtri(jnp.float32(1.0 - TARGET / (N * N)))
    thr = mu + sig * z
    thr_cert = thr + 2e-3 * jnp.maximum(1.0, jnp.abs(thr))

    p_lo, p_hi = _pack_weights()
    pad0 = ((0, NP - N), (0, 0))
    logits, words2d = _score_call(
        thr[None], jnp.pad(s, pad0), jnp.pad(ns, pad0),
        jnp.pad(o, pad0), jnp.pad(no, pad0), p_lo, p_hi)

    words = words2d.reshape(-1)
    lg_flat = logits.reshape(-1)
    cand, vals, cnts = _extract_call(words, lg_flat)

    counts = cnts[:, 0]
    nzs = cnts[:, 1]
    slot = jnp.arange(CAND_PAD)[None, :]
    valid = (slot < jnp.minimum(counts, CAND_CAP)[:, None]).reshape(-1)
    vflat = vals.reshape(-1)
    cflat = cand.reshape(-1)
    sv = jnp.where(valid, jax.nn.sigmoid(vflat), -1.0)
    cert = jnp.sum((valid & (vflat >= thr_cert)).astype(jnp.int32))
    ok = ((cert >= K_PAIRS) & jnp.all(counts <= CAND_CAP)
          & jnp.all(nzs <= NZ_CAP))

    rois0 = rois[0]

    def finish(idx, topv):
        i = idx // NP
        j = idx % NP
        bidx = rois0[i, 0:1]
        boxes_i = rois0[i, 1:5]
        boxes_j = rois0[j, 1:5]
        pairs = jnp.concatenate([bidx, boxes_i, boxes_j], axis=1)
        props = jnp.stack([i, j], axis=1)
        return pairs, props, topv

    def fast():
        topv, pos = lax.top_k(sv, K_PAIRS)
        return finish(cflat[pos], topv)

    def slow():
        r = jnp.arange(NP)
        bad = ((r[:, None] == r[None, :]) | (r[:, None] >= N)
               | (r[None, :] >= N))
        scores = jnp.where(bad, 0.0, jax.nn.sigmoid(logits))
        topv, topi = lax.top_k(scores.reshape(-1), K_PAIRS)
        return finish(topi, topv)

    pairs, props, topv = lax.cond(ok, fast, slow)
    relpn_loss_cls = jnp.array(0.0, dtype=jnp.float32)
    relpn_eval = jnp.zeros((3,), dtype=jnp.float32)
    return (pairs[None], props[None], topv[None], relpn_loss_cls, relpn_eval)
